# baseline (speedup 1.0000x reference)
import functools

import jax
import jax.numpy as jnp
from jax import lax
from jax.experimental import pallas as pl
from jax.experimental.pallas import tpu as pltpu
from jax.experimental.pallas import tpu_sc as plsc

_B = 4096
_D = 1024
_NTOK = 7
_NC = 2
_NS = 16
_NW = _NC * _NS
_EPW = _B // _NW
_NPAIR = _EPW // 2
_PAIR_WORDS = 2 * _NTOK * _D

_CNT = (14, 14, 3, 16, 5, 16, 16)
_OFF = (0, 14, 28, 31, 47, 52, 68)
_VROWS = 84

_TIME_T = tuple(float.fromhex(h) for h in (
    "0x1.322c760000000p-1", "0x1.8db80a0000000p+0", "0x1.8a51e40000000p+1",
    "0x1.6155180000000p+2", "0x1.2d72980000000p+3", "0x1.f4d9080000000p+3",
    "0x1.99be820000000p+4", "0x1.4c2b2a0000000p+5", "0x1.0bcb400000000p+6",
    "0x1.ae53520000000p+6", "0x1.59066a0000000p+7", "0x1.1445b20000000p+8",
    "0x1.ba146a0000000p+8", "0x1.61850c0000000p+9", "0x1.1a9c680000000p+10",
))
_ELO_T = tuple(1200.0 + 100.0 * i for i in range(1, 14))


_I0 = functools.partial(jnp.zeros, (16,), jnp.int32)
_I1 = functools.partial(jnp.full, (16,), 1, jnp.int32)


def _elo_rows(v, off):
    ec = jnp.minimum(jnp.maximum(v, 1200.0), 2500.0)
    lo = _I0()
    one = _I1()
    zero = _I0()
    for a in _ELO_T:
        lo = lo + jnp.where(ec >= a, one, zero)
    lo = jnp.minimum(lo, 12)
    la = 1200.0 + lo.astype(jnp.float32) * 100.0
    tt = jnp.clip((ec - la) / 100.0, 0.0, 1.0)
    return (lo + off) * _D, tt


def _time_rows(v, off):
    u = _I0()
    one = _I1()
    zero = _I0()
    for t in _TIME_T:
        u = u + jnp.where(v >= t, one, zero)
    return (u + off) * _D


def _body(pe, oe, tcc, rt, inc, mt, ot, comb, pos, out,
          tab, stg, posv, f_pe, f_oe, f_rt, f_inc, f_mt, f_ot, tci,
          rowb, tb, sem0, sem1, sem2, sem3):
    sems = (sem0, sem1, sem2, sem3)
    wid = lax.axis_index("s") * _NC + lax.axis_index("c")
    base = wid * _EPW

    pltpu.sync_copy(comb, tab)
    pltpu.sync_copy(pos, posv)
    pltpu.sync_copy(pe.at[pl.ds(base, _EPW)], f_pe)
    pltpu.sync_copy(oe.at[pl.ds(base, _EPW)], f_oe)
    pltpu.sync_copy(rt.at[pl.ds(base, _EPW)], f_rt)
    pltpu.sync_copy(inc.at[pl.ds(base, _EPW)], f_inc)
    pltpu.sync_copy(mt.at[pl.ds(base, _EPW)], f_mt)
    pltpu.sync_copy(ot.at[pl.ds(base, _EPW)], f_ot)
    pltpu.sync_copy(tcc.at[pl.ds(base, _EPW)], tci)

    iota = lax.iota(jnp.int32, 16)

    for t in range(_NTOK):
        @pl.loop(0, _D // 16)
        def _fold_c(c, _t=t):
            pv = posv[pl.ds(_t * _D + c * 16, 16)]

            @plsc.parallel_loop(0, _CNT[_t], unroll=2)
            def _fold_r(r, _pv=pv, _c=c, _o=_OFF[_t]):
                a = pl.ds((_o + r) * _D + _c * 16, 16)
                tab[a] = tab[a] + _pv

    @pl.loop(0, _EPW // 16)
    def _idx(ch):
        s = pl.ds(ch * 16, 16)
        r0, t0 = _elo_rows(f_pe[s], _OFF[0])
        r1, t1 = _elo_rows(f_oe[s], _OFF[1])
        rowb[pl.ds(0 * _EPW + ch * 16, 16)] = r0
        rowb[pl.ds(1 * _EPW + ch * 16, 16)] = r1
        tb[pl.ds(0 * _EPW + ch * 16, 16)] = t0
        tb[pl.ds(1 * _EPW + ch * 16, 16)] = t1
        rowb[pl.ds(2 * _EPW + ch * 16, 16)] = (tci[s] + _OFF[2]) * _D
        rowb[pl.ds(3 * _EPW + ch * 16, 16)] = _time_rows(f_rt[s], _OFF[3])
        iv = f_inc[s]
        zero = _I0()
        ib = (jnp.where(iv == 1.0, _I1(), zero)
              + jnp.where(iv == 2.0, jnp.full((16,), 2, jnp.int32), zero)
              + jnp.where((iv >= 3.0) & (iv < 10.0),
                          jnp.full((16,), 3, jnp.int32), zero)
              + jnp.where(iv >= 10.0, jnp.full((16,), 4, jnp.int32), zero))
        rowb[pl.ds(4 * _EPW + ch * 16, 16)] = (ib + _OFF[4]) * _D
        rowb[pl.ds(5 * _EPW + ch * 16, 16)] = _time_rows(f_mt[s], _OFF[5])
        rowb[pl.ds(6 * _EPW + ch * 16, 16)] = _time_rows(f_ot[s], _OFF[6])

    @pl.loop(0, _EPW, step=4)
    def _elems(e0):
        for b in range(4):
            e = e0 + b
            sem = sems[b]
            cps = [(stg.at[pl.ds((b * _NTOK + tok), 1)],
                    out.at[tok, pl.ds(base + e, 1)])
                   for tok in range(_NTOK)]

            @pl.when(e >= 4)
            def _drain(_cps=cps, _sem=sem):
                for _s, _d in _cps:
                    pltpu.make_async_copy(_s, _d, _sem).wait()

            fe = jnp.zeros((16,), jnp.int32) + e
            bases = [plsc.load_gather(rowb, [fe + (tok * _EPW)])
                     for tok in range(_NTOK)]
            t0 = plsc.load_gather(tb, [fe])
            t1 = plsc.load_gather(tb, [fe + _EPW])
            om0 = 1.0 - t0
            om1 = 1.0 - t1
            u0 = bases[0] + _D
            u1 = bases[1] + _D
            rows = [b * _NTOK + tok for tok in range(_NTOK)]

            @plsc.parallel_loop(0, _D // 16, carry=iota, unroll=8)
            def _chunks(c, col, _b=bases, _u0=u0, _u1=u1, _t0=t0,
                        _t1=t1, _om0=om0, _om1=om1, _rw=rows):
                cw = c * 16
                vl0 = plsc.load_gather(tab, [_b[0] + col])
                vu0 = plsc.load_gather(tab, [_u0 + col])
                stg[_rw[0], pl.ds(cw, 16)] = _om0 * vl0 + _t0 * vu0
                vl1 = plsc.load_gather(tab, [_b[1] + col])
                vu1 = plsc.load_gather(tab, [_u1 + col])
                stg[_rw[1], pl.ds(cw, 16)] = _om1 * vl1 + _t1 * vu1
                for tok in range(2, _NTOK):
                    v = plsc.load_gather(tab, [_b[tok] + col])
                    stg[_rw[tok], pl.ds(cw, 16)] = v
                return col + 16

            for _s, _d in cps:
                pltpu.async_copy(_s, _d, sem)

    for b in range(4):
        e = _EPW - 4 + b
        for tok in range(_NTOK):
            pltpu.make_async_copy(
                stg.at[pl.ds((b * _NTOK + tok), 1)],
                out.at[tok, pl.ds(base + e, 1)], sems[b]).wait()


_sc_call = functools.partial(
    pl.kernel,
    out_type=jax.ShapeDtypeStruct((_NTOK, _B, _D), jnp.float32),
    compiler_params=pltpu.CompilerParams(needs_layout_passes=False),
    mesh=plsc.VectorSubcoreMesh(
        core_axis_name="c", subcore_axis_name="s",
        num_cores=_NC, num_subcores=_NS),
    scratch_types=[
        pltpu.VMEM((_VROWS * _D,), jnp.float32),
        pltpu.VMEM((2 * 2 * _NTOK, _D), jnp.float32),
        pltpu.VMEM((_NTOK * _D,), jnp.float32),
        pltpu.VMEM((_EPW,), jnp.float32),
        pltpu.VMEM((_EPW,), jnp.float32),
        pltpu.VMEM((_EPW,), jnp.float32),
        pltpu.VMEM((_EPW,), jnp.float32),
        pltpu.VMEM((_EPW,), jnp.float32),
        pltpu.VMEM((_EPW,), jnp.float32),
        pltpu.VMEM((_EPW,), jnp.int32),
        pltpu.VMEM((_NTOK * _EPW,), jnp.int32),
        pltpu.VMEM((2 * _EPW,), jnp.float32),
        pltpu.SemaphoreType.DMA,
        pltpu.SemaphoreType.DMA,
        pltpu.SemaphoreType.DMA,
        pltpu.SemaphoreType.DMA,
    ],
)(_body)


def kernel(player_elo, opp_elo, tc_cat, remaining_time, increment,
           my_last_time, opp_last_time, elo_emb, opp_elo_emb, tc_emb,
           urgency_emb, inc_emb, my_time_emb, opp_time_emb, pos_emb):
    comb = jnp.concatenate(
        [elo_emb, opp_elo_emb, tc_emb, urgency_emb, inc_emb,
         my_time_emb, opp_time_emb], axis=0).reshape(-1)
    out = _sc_call(player_elo, opp_elo, tc_cat.astype(jnp.int32),
                   remaining_time, increment, my_last_time, opp_last_time,
                   comb, pos_emb.reshape(-1))
    return out.transpose(1, 0, 2)


# speedup vs baseline: 1.0251x; 1.0251x over previous
import functools

import jax
import jax.numpy as jnp
from jax import lax
from jax.experimental import pallas as pl
from jax.experimental.pallas import tpu as pltpu
from jax.experimental.pallas import tpu_sc as plsc

_B = 4096
_D = 1024
_NTOK = 7
_NC = 2
_NS = 16
_NW = _NC * _NS
_EPW = _B // _NW
_NPAIR = _EPW // 2
_PAIR_WORDS = 2 * _NTOK * _D

_CNT = (14, 14, 3, 16, 5, 16, 16)
_OFF = (0, 14, 28, 31, 47, 52, 68)
_VROWS = 84

_TIME_T = tuple(float.fromhex(h) for h in (
    "0x1.322c760000000p-1", "0x1.8db80a0000000p+0", "0x1.8a51e40000000p+1",
    "0x1.6155180000000p+2", "0x1.2d72980000000p+3", "0x1.f4d9080000000p+3",
    "0x1.99be820000000p+4", "0x1.4c2b2a0000000p+5", "0x1.0bcb400000000p+6",
    "0x1.ae53520000000p+6", "0x1.59066a0000000p+7", "0x1.1445b20000000p+8",
    "0x1.ba146a0000000p+8", "0x1.61850c0000000p+9", "0x1.1a9c680000000p+10",
))
_ELO_T = tuple(1200.0 + 100.0 * i for i in range(1, 14))


_I0 = functools.partial(jnp.zeros, (16,), jnp.int32)
_I1 = functools.partial(jnp.full, (16,), 1, jnp.int32)


def _elo_rows(v, off):
    ec = jnp.minimum(jnp.maximum(v, 1200.0), 2500.0)
    lo = _I0()
    one = _I1()
    zero = _I0()
    for a in _ELO_T:
        lo = lo + jnp.where(ec >= a, one, zero)
    lo = jnp.minimum(lo, 12)
    la = 1200.0 + lo.astype(jnp.float32) * 100.0
    tt = jnp.clip((ec - la) / 100.0, 0.0, 1.0)
    return (lo + off) * _D, tt


def _time_rows(v, off):
    u = _I0()
    one = _I1()
    zero = _I0()
    for t in _TIME_T:
        u = u + jnp.where(v >= t, one, zero)
    return (u + off) * _D


def _body(pe, oe, tcc, rt, inc, mt, ot, comb, pos, out,
          tab, stg, posv, f_pe, f_oe, f_rt, f_inc, f_mt, f_ot, tci,
          rowb, tb, sem0, sem1):
    wid = lax.axis_index("s") * _NC + lax.axis_index("c")
    base = wid * _EPW

    pltpu.sync_copy(comb, tab)
    pltpu.sync_copy(pos, posv)
    pltpu.sync_copy(pe.at[pl.ds(base, _EPW)], f_pe)
    pltpu.sync_copy(oe.at[pl.ds(base, _EPW)], f_oe)
    pltpu.sync_copy(rt.at[pl.ds(base, _EPW)], f_rt)
    pltpu.sync_copy(inc.at[pl.ds(base, _EPW)], f_inc)
    pltpu.sync_copy(mt.at[pl.ds(base, _EPW)], f_mt)
    pltpu.sync_copy(ot.at[pl.ds(base, _EPW)], f_ot)
    pltpu.sync_copy(tcc.at[pl.ds(base, _EPW)], tci)

    iota = lax.iota(jnp.int32, 16)

    for t in range(_NTOK):
        @pl.loop(0, _D // 16)
        def _fold_c(c, _t=t):
            pv = posv[pl.ds(_t * _D + c * 16, 16)]

            @plsc.parallel_loop(0, _CNT[_t], unroll=2)
            def _fold_r(r, _pv=pv, _c=c, _o=_OFF[_t]):
                a = pl.ds((_o + r) * _D + _c * 16, 16)
                tab[a] = tab[a] + _pv

    @pl.loop(0, _EPW // 16)
    def _idx(ch):
        s = pl.ds(ch * 16, 16)
        r0, t0 = _elo_rows(f_pe[s], _OFF[0])
        r1, t1 = _elo_rows(f_oe[s], _OFF[1])
        rowb[pl.ds(0 * _EPW + ch * 16, 16)] = r0
        rowb[pl.ds(1 * _EPW + ch * 16, 16)] = r1
        tb[pl.ds(0 * _EPW + ch * 16, 16)] = t0
        tb[pl.ds(1 * _EPW + ch * 16, 16)] = t1
        rowb[pl.ds(2 * _EPW + ch * 16, 16)] = (tci[s] + _OFF[2]) * _D
        rowb[pl.ds(3 * _EPW + ch * 16, 16)] = _time_rows(f_rt[s], _OFF[3])
        iv = f_inc[s]
        zero = _I0()
        ib = (jnp.where(iv == 1.0, _I1(), zero)
              + jnp.where(iv == 2.0, jnp.full((16,), 2, jnp.int32), zero)
              + jnp.where((iv >= 3.0) & (iv < 10.0),
                          jnp.full((16,), 3, jnp.int32), zero)
              + jnp.where(iv >= 10.0, jnp.full((16,), 4, jnp.int32), zero))
        rowb[pl.ds(4 * _EPW + ch * 16, 16)] = (ib + _OFF[4]) * _D
        rowb[pl.ds(5 * _EPW + ch * 16, 16)] = _time_rows(f_mt[s], _OFF[5])
        rowb[pl.ds(6 * _EPW + ch * 16, 16)] = _time_rows(f_ot[s], _OFF[6])

    @pl.loop(0, _NPAIR, step=2)
    def _pairs(g):
        for b in range(2):
            p = g + b
            sem = sem0 if b == 0 else sem1
            cps = [(stg.at[pl.ds((b * _NTOK + tok) * 2, 2)],
                    out.at[tok, pl.ds(base + p * 2, 2)])
                   for tok in range(_NTOK)]

            @pl.when(p >= 2)
            def _drain(_cps=cps, _sem=sem):
                for _s, _d in _cps:
                    pltpu.make_async_copy(_s, _d, _sem).wait()

            for es in range(2):
                e = p * 2 + es
                fe = jnp.zeros((16,), jnp.int32) + e
                bases = [plsc.load_gather(rowb, [fe + (tok * _EPW)])
                         for tok in range(_NTOK)]
                t0 = plsc.load_gather(tb, [fe])
                t1 = plsc.load_gather(tb, [fe + _EPW])
                om0 = 1.0 - t0
                om1 = 1.0 - t1
                u0 = bases[0] + _D
                u1 = bases[1] + _D

                rows = [(b * _NTOK + tok) * 2 + es
                        for tok in range(_NTOK)]

                @plsc.parallel_loop(0, _D // 16, carry=iota, unroll=8)
                def _chunks(c, col, _b=bases, _u0=u0, _u1=u1, _t0=t0,
                            _t1=t1, _om0=om0, _om1=om1, _rw=rows):
                    cw = c * 16
                    vl0 = plsc.load_gather(tab, [_b[0] + col])
                    vu0 = plsc.load_gather(tab, [_u0 + col])
                    stg[_rw[0], pl.ds(cw, 16)] = _om0 * vl0 + _t0 * vu0
                    vl1 = plsc.load_gather(tab, [_b[1] + col])
                    vu1 = plsc.load_gather(tab, [_u1 + col])
                    stg[_rw[1], pl.ds(cw, 16)] = _om1 * vl1 + _t1 * vu1
                    for tok in range(2, _NTOK):
                        v = plsc.load_gather(tab, [_b[tok] + col])
                        stg[_rw[tok], pl.ds(cw, 16)] = v
                    return col + 16

            for _s, _d in cps:
                pltpu.async_copy(_s, _d, sem)

    for b in range(2):
        p = _NPAIR - 2 + b
        sem = sem0 if b == 0 else sem1
        for tok in range(_NTOK):
            pltpu.make_async_copy(
                stg.at[pl.ds((b * _NTOK + tok) * 2, 2)],
                out.at[tok, pl.ds(base + p * 2, 2)], sem).wait()


_sc_call = functools.partial(
    pl.kernel,
    out_type=jax.ShapeDtypeStruct((_NTOK, _B, _D), jnp.float32),
    compiler_params=pltpu.CompilerParams(needs_layout_passes=False),
    mesh=plsc.VectorSubcoreMesh(
        core_axis_name="c", subcore_axis_name="s",
        num_cores=_NC, num_subcores=_NS),
    scratch_types=[
        pltpu.VMEM((_VROWS * _D,), jnp.float32),
        pltpu.VMEM((2 * 2 * _NTOK, _D), jnp.float32),
        pltpu.VMEM((_NTOK * _D,), jnp.float32),
        pltpu.VMEM((_EPW,), jnp.float32),
        pltpu.VMEM((_EPW,), jnp.float32),
        pltpu.VMEM((_EPW,), jnp.float32),
        pltpu.VMEM((_EPW,), jnp.float32),
        pltpu.VMEM((_EPW,), jnp.float32),
        pltpu.VMEM((_EPW,), jnp.float32),
        pltpu.VMEM((_EPW,), jnp.int32),
        pltpu.VMEM((_NTOK * _EPW,), jnp.int32),
        pltpu.VMEM((2 * _EPW,), jnp.float32),
        pltpu.SemaphoreType.DMA,
        pltpu.SemaphoreType.DMA,
    ],
)(_body)


def kernel(player_elo, opp_elo, tc_cat, remaining_time, increment,
           my_last_time, opp_last_time, elo_emb, opp_elo_emb, tc_emb,
           urgency_emb, inc_emb, my_time_emb, opp_time_emb, pos_emb):
    comb = jnp.concatenate(
        [elo_emb, opp_elo_emb, tc_emb, urgency_emb, inc_emb,
         my_time_emb, opp_time_emb], axis=0).reshape(-1)
    out = _sc_call(player_elo, opp_elo, tc_cat.astype(jnp.int32),
                   remaining_time, increment, my_last_time, opp_last_time,
                   comb, pos_emb.reshape(-1))
    return out.transpose(1, 0, 2)
